# Initial kernel scaffold; baseline (speedup 1.0000x reference)
import jax
import jax.numpy as jnp
from jax.experimental import pallas as pl


def kernel(x, edge_index, edge_attr, batch, params):
    raise NotImplementedError("write your pallas kernel here")


# speedup vs baseline: 2.1784x; 2.1784x over previous
import functools

import jax
import jax.numpy as jnp
from jax import lax
from jax.experimental import pallas as pl
from jax.experimental.pallas import tpu as pltpu
from jax.experimental.pallas import tpu_sc as plsc

N = 50000
E = 800000
EPAD = 819200
NG = 64
HN = 64

NSC = 2
NTILE = 16
EPT = EPAD // NTILE
CE = 256
NCH = EPT // CE
IPR = 128
NB = CE // IPR
NPAD = 50048
TILE_N = NPAD // NTILE

EB = 2048
EP4 = EPAD // 4
TC1_GRID = EPAD // EB
LAST_EA_BLOCK = (E - 1) // EB

XB = 1000
TC2_GRID = N // XB


def _tc1_body(ea_ref, w_ref, b_ref, out_ref):
    i = pl.program_id(0)
    ea = ea_ref[...]
    acts = jnp.dot(ea, w_ref[0], preferred_element_type=jnp.float32)
    acts = jnp.maximum(acts + b_ref[0], 0.0)
    rows = i * EB + lax.broadcasted_iota(jnp.int32, (EB, 32), 0)
    acts = jnp.where(rows < E, acts, 0.0)
    out_ref[0] = jnp.concatenate(
        [acts[m * 512:(m + 1) * 512, :] for m in range(4)], axis=1)


def _tc1_call(edge_attr, w2, b2):
    return pl.pallas_call(
        _tc1_body,
        grid=(TC1_GRID, NSC),
        in_specs=[
            pl.BlockSpec((EB, 16), lambda i, h: (jnp.minimum(i, LAST_EA_BLOCK), 0)),
            pl.BlockSpec((1, 16, 32), lambda i, h: (h, 0, 0)),
            pl.BlockSpec((1, 1, 32), lambda i, h: (h, 0, 0)),
        ],
        out_specs=pl.BlockSpec((1, 512, 128), lambda i, h: (h, i, 0)),
        out_shape=jax.ShapeDtypeStruct((NSC, EP4, 128), jnp.float32),
    )(edge_attr, w2, b2)


def _sc_body(he, dstv, zeros, agg, table, rows0, rows1, idx0, idx1,
             sem_r0, sem_r1, sem_i0, sem_i1, sem_s, sem_init, sem_out):
    c = lax.axis_index("c")
    s = lax.axis_index("s")
    ebase = s * EPT
    rbase = s * (EPT // IPR)

    rows = (rows0, rows1)
    idx = (idx0, idx1)
    sem_r = (sem_r0, sem_r1)
    sem_i = (sem_i0, sem_i1)

    def start(j, b):
        pr = (ebase + j * CE) // 4
        for m in range(4):
            pltpu.async_copy(
                he.at[c, pl.ds(pr, CE // 4), pl.ds(32 * m, 32)],
                rows[b].at[pl.ds(m * (CE // 4), CE // 4), :],
                sem_r[b],
            )
        pltpu.async_copy(dstv.at[pl.ds(rbase + j * NB, NB), :], idx[b], sem_i[b])

    start(0, 0)
    start(1, 1)
    pltpu.async_copy(
        zeros.at[pl.ds(s * TILE_N, TILE_N), :],
        table.at[pl.ds(s * TILE_N, TILE_N), :],
        sem_init,
    ).wait()
    plsc.subcore_barrier()

    def chunk(j, b):
        pr = (ebase + j * CE) // 4
        for m in range(4):
            pltpu.make_async_copy(
                he.at[c, pl.ds(pr, CE // 4), pl.ds(32 * m, 32)],
                rows[b].at[pl.ds(m * (CE // 4), CE // 4), :],
                sem_r[b],
            ).wait()
        pltpu.make_async_copy(
            dstv.at[pl.ds(rbase + j * NB, NB), :], idx[b], sem_i[b]
        ).wait()
        descs = [
            pltpu.async_copy(
                rows[b].at[pl.ds(k * IPR, IPR), :],
                table.at[idx[b].at[k]],
                sem_s,
                add=True,
            )
            for k in range(NB)
        ]
        for d in descs:
            d.wait()

        @pl.when(j + 2 < NCH)
        def _():
            start(j + 2, b)

    def body(jj, carry):
        chunk(2 * jj, 0)
        chunk(2 * jj + 1, 1)
        return carry

    lax.fori_loop(0, NCH // 2, body, 0)

    plsc.subcore_barrier()
    pltpu.async_copy(
        table.at[pl.ds(s * TILE_N, TILE_N), :],
        agg.at[c, pl.ds(s * TILE_N, TILE_N), :],
        sem_out,
    ).wait()


def _sc_call(he, dstv, zeros):
    mesh = plsc.VectorSubcoreMesh(
        core_axis_name="c", subcore_axis_name="s",
        num_cores=NSC, num_subcores=NTILE,
    )
    fn = functools.partial(
        pl.kernel,
        mesh=mesh,
        compiler_params=pltpu.CompilerParams(use_tc_tiling_on_sc=False),
        out_type=jax.ShapeDtypeStruct((NSC, NPAD, 32), jnp.float32),
        scratch_types=[
            pltpu.VMEM_SHARED((NPAD, 32), jnp.float32),
            pltpu.VMEM((CE, 32), jnp.float32),
            pltpu.VMEM((CE, 32), jnp.float32),
            pltpu.VMEM((NB, IPR), jnp.int32),
            pltpu.VMEM((NB, IPR), jnp.int32),
            pltpu.SemaphoreType.DMA,
            pltpu.SemaphoreType.DMA,
            pltpu.SemaphoreType.DMA,
            pltpu.SemaphoreType.DMA,
            pltpu.SemaphoreType.DMA,
            pltpu.SemaphoreType.DMA,
            pltpu.SemaphoreType.DMA,
        ],
    )(_sc_body)
    return fn(he, dstv, zeros)


def _tc2_body(x_ref, agg_ref, batch_ref, wnn_ref, bnn_ref, wun_ref, bun_ref,
              wpred_ref, bpred_ref, out_ref, sums_ref, cnt_ref):
    i = pl.program_id(0)
    hn = jnp.dot(x_ref[...], wnn_ref[...], preferred_element_type=jnp.float32)
    hn = jnp.maximum(hn + bnn_ref[...], 0.0)
    aggb = jnp.concatenate([agg_ref[0], agg_ref[1]], axis=1)
    h = (jnp.dot(hn, wun_ref[0:HN, :], preferred_element_type=jnp.float32)
         + jnp.dot(aggb, wun_ref[HN:2 * HN, :], preferred_element_type=jnp.float32)
         + bun_ref[...])
    h = jnp.maximum(h, 0.0)
    bb = batch_ref[0, 0, :]
    onehot = (bb[:, None] == lax.broadcasted_iota(jnp.int32, (XB, NG), 1))
    onehot = onehot.astype(jnp.float32)
    psum = lax.dot_general(onehot, h, (((0,), (0,)), ((), ())),
                           preferred_element_type=jnp.float32)
    pcnt = lax.dot_general(onehot, jnp.ones((XB, 1), jnp.float32),
                           (((0,), (0,)), ((), ())),
                           preferred_element_type=jnp.float32)

    @pl.when(i == 0)
    def _():
        sums_ref[...] = psum
        cnt_ref[...] = pcnt

    @pl.when(i > 0)
    def _():
        sums_ref[...] += psum
        cnt_ref[...] += pcnt

    @pl.when(i == TC2_GRID - 1)
    def _():
        hg = sums_ref[...] / jnp.maximum(cnt_ref[...], 1.0)
        out_ref[...] = (jnp.dot(hg, wpred_ref[...],
                                preferred_element_type=jnp.float32)
                        + bpred_ref[...])


def _tc2_call(x, agg, batch3, w_nn, b_nn, w_un, b_un, w_pred, b_pred):
    return pl.pallas_call(
        _tc2_body,
        grid=(TC2_GRID,),
        in_specs=[
            pl.BlockSpec((XB, 128), lambda i: (i, 0)),
            pl.BlockSpec((NSC, XB, 32), lambda i: (0, i, 0)),
            pl.BlockSpec((1, 1, XB), lambda i: (i, 0, 0)),
            pl.BlockSpec((128, HN), lambda i: (0, 0)),
            pl.BlockSpec((1, HN), lambda i: (0, 0)),
            pl.BlockSpec((2 * HN, HN), lambda i: (0, 0)),
            pl.BlockSpec((1, HN), lambda i: (0, 0)),
            pl.BlockSpec((HN, 1), lambda i: (0, 0)),
            pl.BlockSpec((1, 1), lambda i: (0, 0)),
        ],
        out_specs=pl.BlockSpec((NG, 1), lambda i: (0, 0)),
        out_shape=jax.ShapeDtypeStruct((NG, 1), jnp.float32),
        scratch_shapes=[
            pltpu.VMEM((NG, NG), jnp.float32),
            pltpu.VMEM((NG, 1), jnp.float32),
        ],
    )(x, agg, batch3, w_nn, b_nn, w_un, b_un, w_pred, b_pred)


def kernel(x, edge_index, edge_attr, batch, params):
    p0 = params['layers'][0]
    dst = edge_index[1].astype(jnp.int32)
    dst2d = jnp.pad(dst.reshape(E // IPR, IPR), ((0, (EPAD - E) // IPR), (0, 0)))
    dstv = (dst2d.reshape(TC1_GRID, 4, 8, 64)
            .transpose(0, 2, 1, 3)
            .reshape(EPAD // IPR, IPR))

    w2 = jnp.stack([p0['W_en'][:, :32], p0['W_en'][:, 32:]])
    b2 = p0['b_en'].reshape(NSC, 1, 32)
    he = _tc1_call(edge_attr, w2, b2)
    zeros = jnp.zeros((NPAD, 32), jnp.float32)
    agg = _sc_call(he, dstv, zeros)

    batch3 = batch.astype(jnp.int32).reshape(TC2_GRID, 1, XB)
    out = _tc2_call(
        x, agg, batch3,
        p0['W_nn'], p0['b_nn'].reshape(1, HN),
        p0['W_un'], p0['b_un'].reshape(1, HN),
        params['W_pred'], params['b_pred'].reshape(1, 1),
    )
    return out.reshape(-1)


# speedup vs baseline: 2.7188x; 1.2481x over previous
import functools

import jax
import jax.numpy as jnp
from jax import lax
from jax.experimental import pallas as pl
from jax.experimental.pallas import tpu as pltpu
from jax.experimental.pallas import tpu_sc as plsc

N = 50000
E = 800000
EPAD = 819200
NG = 64
HN = 64

NSC = 2
NTILE = 16
EPT = EPAD // NTILE
CE = 256
NCH = EPT // CE
IPR = 128
NB = CE // IPR
NPAD = 50048
TILE_N = NPAD // NTILE

EB = 4096
EBR = EB // 4
EP4 = EPAD // 4
TC1_GRID = EPAD // EB
LAST_EA_BLOCK = (E - 1) // EB
FIRST_MASKED_BLOCK = E // EB

XB = 1000
TC2_GRID = N // XB


def _tc1_body(ea_ref, w_ref, b_ref, out_ref):
    i = pl.program_id(0)

    def acts_from(ea):
        acc = jnp.dot(ea[0:EBR, :], w_ref[0, 0],
                      preferred_element_type=jnp.float32)
        for m in range(1, 4):
            acc = acc + jnp.dot(ea[m * EBR:(m + 1) * EBR, :], w_ref[0, m],
                                preferred_element_type=jnp.float32)
        return jnp.maximum(acc + b_ref[0], 0.0)

    @pl.when(i < FIRST_MASKED_BLOCK)
    def _():
        out_ref[0] = acts_from(ea_ref[...])

    @pl.when(i >= FIRST_MASKED_BLOCK)
    def _():
        in_rows = i * EB + lax.broadcasted_iota(jnp.int32, (EB, 16), 0)
        ea = jnp.where(in_rows < E, ea_ref[...], 0.0)
        acts = acts_from(ea)
        lanes = lax.broadcasted_iota(jnp.int32, (EBR, 128), 1)
        rows = lax.broadcasted_iota(jnp.int32, (EBR, 128), 0)
        edge = i * EB + (lanes // 32) * EBR + rows
        out_ref[0] = jnp.where(edge < E, acts, 0.0)


def _tc1_call(edge_attr, w2, b2):
    return pl.pallas_call(
        _tc1_body,
        grid=(TC1_GRID, NSC),
        in_specs=[
            pl.BlockSpec((EB, 16), lambda i, h: (jnp.minimum(i, LAST_EA_BLOCK), 0)),
            pl.BlockSpec((1, 4, 16, 128), lambda i, h: (h, 0, 0, 0)),
            pl.BlockSpec((1, 1, 128), lambda i, h: (h, 0, 0)),
        ],
        out_specs=pl.BlockSpec((1, EBR, 128), lambda i, h: (h, i, 0)),
        out_shape=jax.ShapeDtypeStruct((NSC, EP4, 128), jnp.float32),
    )(edge_attr, w2, b2)


def _sc_body(he, dstv, zeros, agg, table, rows0, rows1, idx0, idx1,
             sem_r0, sem_r1, sem_i0, sem_i1, sem_s, sem_init, sem_out):
    c = lax.axis_index("c")
    s = lax.axis_index("s")
    ebase = s * EPT
    rbase = s * (EPT // IPR)

    rows = (rows0, rows1)
    idx = (idx0, idx1)
    sem_r = (sem_r0, sem_r1)
    sem_i = (sem_i0, sem_i1)

    def start(j, b):
        pr = (ebase + j * CE) // 4
        for m in range(4):
            pltpu.async_copy(
                he.at[c, pl.ds(pr, CE // 4), pl.ds(32 * m, 32)],
                rows[b].at[pl.ds(m * (CE // 4), CE // 4), :],
                sem_r[b],
            )
        pltpu.async_copy(dstv.at[pl.ds(rbase + j * NB, NB), :], idx[b], sem_i[b])

    start(0, 0)
    start(1, 1)
    pltpu.async_copy(
        zeros.at[pl.ds(s * TILE_N, TILE_N), :],
        table.at[pl.ds(s * TILE_N, TILE_N), :],
        sem_init,
    ).wait()
    plsc.subcore_barrier()

    def chunk(j, b):
        pr = (ebase + j * CE) // 4
        for m in range(4):
            pltpu.make_async_copy(
                he.at[c, pl.ds(pr, CE // 4), pl.ds(32 * m, 32)],
                rows[b].at[pl.ds(m * (CE // 4), CE // 4), :],
                sem_r[b],
            ).wait()
        pltpu.make_async_copy(
            dstv.at[pl.ds(rbase + j * NB, NB), :], idx[b], sem_i[b]
        ).wait()
        descs = [
            pltpu.async_copy(
                rows[b].at[pl.ds(k * IPR, IPR), :],
                table.at[idx[b].at[k]],
                sem_s,
                add=True,
            )
            for k in range(NB)
        ]
        for d in descs:
            d.wait()

        @pl.when(j + 2 < NCH)
        def _():
            start(j + 2, b)

    def body(jj, carry):
        chunk(2 * jj, 0)
        chunk(2 * jj + 1, 1)
        return carry

    lax.fori_loop(0, NCH // 2, body, 0)

    plsc.subcore_barrier()
    pltpu.async_copy(
        table.at[pl.ds(s * TILE_N, TILE_N), :],
        agg.at[c, pl.ds(s * TILE_N, TILE_N), :],
        sem_out,
    ).wait()


def _sc_call(he, dstv, zeros):
    mesh = plsc.VectorSubcoreMesh(
        core_axis_name="c", subcore_axis_name="s",
        num_cores=NSC, num_subcores=NTILE,
    )
    fn = functools.partial(
        pl.kernel,
        mesh=mesh,
        compiler_params=pltpu.CompilerParams(use_tc_tiling_on_sc=False),
        out_type=jax.ShapeDtypeStruct((NSC, NPAD, 32), jnp.float32),
        scratch_types=[
            pltpu.VMEM_SHARED((NPAD, 32), jnp.float32),
            pltpu.VMEM((CE, 32), jnp.float32),
            pltpu.VMEM((CE, 32), jnp.float32),
            pltpu.VMEM((NB, IPR), jnp.int32),
            pltpu.VMEM((NB, IPR), jnp.int32),
            pltpu.SemaphoreType.DMA,
            pltpu.SemaphoreType.DMA,
            pltpu.SemaphoreType.DMA,
            pltpu.SemaphoreType.DMA,
            pltpu.SemaphoreType.DMA,
            pltpu.SemaphoreType.DMA,
            pltpu.SemaphoreType.DMA,
        ],
    )(_sc_body)
    return fn(he, dstv, zeros)


def _tc2_body(x_ref, agg_ref, batch_ref, wnn_ref, bnn_ref, wun_ref, bun_ref,
              wpred_ref, bpred_ref, out_ref, sums_ref, cnt_ref):
    i = pl.program_id(0)
    hn = jnp.dot(x_ref[...], wnn_ref[...], preferred_element_type=jnp.float32)
    hn = jnp.maximum(hn + bnn_ref[...], 0.0)
    aggb = jnp.concatenate([agg_ref[0], agg_ref[1]], axis=1)
    h = (jnp.dot(hn, wun_ref[0:HN, :], preferred_element_type=jnp.float32)
         + jnp.dot(aggb, wun_ref[HN:2 * HN, :], preferred_element_type=jnp.float32)
         + bun_ref[...])
    h = jnp.maximum(h, 0.0)
    bb = batch_ref[0, 0, :]
    onehot = (bb[:, None] == lax.broadcasted_iota(jnp.int32, (XB, NG), 1))
    onehot = onehot.astype(jnp.float32)
    psum = lax.dot_general(onehot, h, (((0,), (0,)), ((), ())),
                           preferred_element_type=jnp.float32)
    pcnt = lax.dot_general(onehot, jnp.ones((XB, 1), jnp.float32),
                           (((0,), (0,)), ((), ())),
                           preferred_element_type=jnp.float32)

    @pl.when(i == 0)
    def _():
        sums_ref[...] = psum
        cnt_ref[...] = pcnt

    @pl.when(i > 0)
    def _():
        sums_ref[...] += psum
        cnt_ref[...] += pcnt

    @pl.when(i == TC2_GRID - 1)
    def _():
        hg = sums_ref[...] / jnp.maximum(cnt_ref[...], 1.0)
        out_ref[...] = (jnp.dot(hg, wpred_ref[...],
                                preferred_element_type=jnp.float32)
                        + bpred_ref[...])


def _tc2_call(x, agg, batch3, w_nn, b_nn, w_un, b_un, w_pred, b_pred):
    return pl.pallas_call(
        _tc2_body,
        grid=(TC2_GRID,),
        in_specs=[
            pl.BlockSpec((XB, 128), lambda i: (i, 0)),
            pl.BlockSpec((NSC, XB, 32), lambda i: (0, i, 0)),
            pl.BlockSpec((1, 1, XB), lambda i: (i, 0, 0)),
            pl.BlockSpec((128, HN), lambda i: (0, 0)),
            pl.BlockSpec((1, HN), lambda i: (0, 0)),
            pl.BlockSpec((2 * HN, HN), lambda i: (0, 0)),
            pl.BlockSpec((1, HN), lambda i: (0, 0)),
            pl.BlockSpec((HN, 1), lambda i: (0, 0)),
            pl.BlockSpec((1, 1), lambda i: (0, 0)),
        ],
        out_specs=pl.BlockSpec((NG, 1), lambda i: (0, 0)),
        out_shape=jax.ShapeDtypeStruct((NG, 1), jnp.float32),
        scratch_shapes=[
            pltpu.VMEM((NG, NG), jnp.float32),
            pltpu.VMEM((NG, 1), jnp.float32),
        ],
    )(x, agg, batch3, w_nn, b_nn, w_un, b_un, w_pred, b_pred)


def kernel(x, edge_index, edge_attr, batch, params):
    p0 = params['layers'][0]
    dst = edge_index[1].astype(jnp.int32)
    dst2d = jnp.pad(dst.reshape(E // IPR, IPR), ((0, (EPAD - E) // IPR), (0, 0)))
    dstv = (dst2d.reshape(TC1_GRID, 4, 16, 64)
            .transpose(0, 2, 1, 3)
            .reshape(EPAD // IPR, IPR))

    w4 = jnp.zeros((NSC, 4, 16, 128), jnp.float32)
    for c in range(NSC):
        half = p0['W_en'][:, c * 32:(c + 1) * 32]
        for m in range(4):
            w4 = w4.at[c, m, :, m * 32:(m + 1) * 32].set(half)
    b2 = jnp.stack([jnp.tile(p0['b_en'][c * 32:(c + 1) * 32], 4).reshape(1, 128)
                    for c in range(NSC)])
    he = _tc1_call(edge_attr, w4, b2)
    zeros = jnp.zeros((NPAD, 32), jnp.float32)
    agg = _sc_call(he, dstv, zeros)

    batch3 = batch.astype(jnp.int32).reshape(TC2_GRID, 1, XB)
    out = _tc2_call(
        x, agg, batch3,
        p0['W_nn'], p0['b_nn'].reshape(1, HN),
        p0['W_un'], p0['b_un'].reshape(1, HN),
        params['W_pred'], params['b_pred'].reshape(1, 1),
    )
    return out.reshape(-1)
